# Initial kernel scaffold; baseline (speedup 1.0000x reference)
import jax
import jax.numpy as jnp
from jax.experimental import pallas as pl


def kernel(img, tf1, tf2, tf3):
    raise NotImplementedError("write your pallas kernel here")


# speedup vs baseline: 1108.0079x; 1108.0079x over previous
import functools

import jax
import jax.numpy as jnp
from jax import lax
from jax.experimental import pallas as pl
from jax.experimental.pallas import tpu as pltpu
from jax.experimental.pallas import tpu_sc as plsc

NC = 2
NS = 16
L = 16
NW = NC * NS

P = 48
PLANE = 512 * 512
SEG = PLANE // NW
NLUT = 256

def _mesh():
    return plsc.VectorSubcoreMesh(
        core_axis_name="c", subcore_axis_name="s", num_cores=NC, num_subcores=NS
    )


def _body(img_h, t1_h, t2_h, t3_h, o1_h, o2_h, o3_h,
          t1_v, t2_v, t3_v, in_v, o1_v, o2_v, o3_v, sem):
    wid = lax.axis_index("s") * NC + lax.axis_index("c")
    pltpu.async_copy(t1_h, t1_v, sem).wait()
    pltpu.async_copy(t2_h, t2_v, sem).wait()
    pltpu.async_copy(t3_h, t3_v, sem).wait()

    def plane_body(p, _):
        base = pl.multiple_of(p * PLANE + wid * SEG, SEG)
        pltpu.async_copy(img_h.at[pl.ds(base, SEG)], in_v, sem).wait()
        magic = (p * NLUT).astype(jnp.float32) + jnp.float32(2.0**23)
        magic_v = jnp.full((L,), 0.0, jnp.float32) + magic

        def vec_body(i, _):
            o = i * L
            x = in_v[pl.ds(o, L)]
            f = x * jnp.float32(255.0) + magic_v
            idx = lax.bitcast_convert_type(f, jnp.int32) & jnp.int32(0x3FFF)
            o1_v[pl.ds(o, L)] = plsc.load_gather(t1_v, [idx])
            o2_v[pl.ds(o, L)] = plsc.load_gather(t2_v, [idx])
            o3_v[pl.ds(o, L)] = plsc.load_gather(t3_v, [idx])
            return 0

        lax.fori_loop(0, SEG // L, vec_body, 0)
        pltpu.async_copy(o1_v, o1_h.at[pl.ds(base, SEG)], sem).wait()
        pltpu.async_copy(o2_v, o2_h.at[pl.ds(base, SEG)], sem).wait()
        pltpu.async_copy(o3_v, o3_h.at[pl.ds(base, SEG)], sem).wait()
        return 0

    lax.fori_loop(0, P, plane_body, 0)


@functools.partial(jax.jit)
def _run(imgf, t1, t2, t3):
    out_t = tuple(jax.ShapeDtypeStruct((P * PLANE,), jnp.float32) for _ in range(3))
    scratch = [
        pltpu.VMEM((P * NLUT,), jnp.float32),
        pltpu.VMEM((P * NLUT,), jnp.float32),
        pltpu.VMEM((P * NLUT,), jnp.float32),
        pltpu.VMEM((SEG,), jnp.float32),
        pltpu.VMEM((SEG,), jnp.float32),
        pltpu.VMEM((SEG,), jnp.float32),
        pltpu.VMEM((SEG,), jnp.float32),
        pltpu.SemaphoreType.DMA,
    ]
    f = pl.kernel(
        _body, out_type=out_t, mesh=_mesh(), scratch_types=scratch,
        compiler_params=pltpu.CompilerParams(needs_layout_passes=False),
    )
    return f(imgf, t1, t2, t3)


def kernel(img, tf1, tf2, tf3):
    B, C, H, W = img.shape
    o1, o2, o3 = _run(
        img.reshape(P * PLANE),
        tf1.reshape(P * NLUT),
        tf2.reshape(P * NLUT),
        tf3.reshape(P * NLUT),
    )
    shp = (B, C, H, W)
    return (o1.reshape(shp), o2.reshape(shp), o3.reshape(shp))


# speedup vs baseline: 2015.6087x; 1.8191x over previous
import functools

import jax
import jax.numpy as jnp
from jax import lax
from jax.experimental import pallas as pl
from jax.experimental.pallas import tpu as pltpu
from jax.experimental.pallas import tpu_sc as plsc

NC = 2
NS = 16
L = 16
NW = NC * NS

P = 48
PLANE = 512 * 512
SEG = PLANE // NW
NLUT = 256
VITER = SEG // L


def _mesh():
    return plsc.VectorSubcoreMesh(
        core_axis_name="c", subcore_axis_name="s", num_cores=NC, num_subcores=NS
    )


def _body(img_h, t1_h, t2_h, t3_h, o1_h, o2_h, o3_h,
          t1_v, t2_v, t3_v, in_v, o1_v, o2_v, o3_v, sem_in, sem_out):
    wid = lax.axis_index("s") * NC + lax.axis_index("c")
    off = wid * SEG
    pltpu.async_copy(t1_h, t1_v, sem_in).wait()
    pltpu.async_copy(t2_h, t2_v, sem_in).wait()
    pltpu.async_copy(t3_h, t3_v, sem_in).wait()

    def base(p):
        return pl.multiple_of(p * PLANE + off, SEG)

    def issue_in(p, k):
        pltpu.async_copy(img_h.at[pl.ds(base(p), SEG)], in_v.at[k], sem_in)

    def wait_in(p, k):
        pltpu.make_async_copy(img_h.at[pl.ds(base(p), SEG)], in_v.at[k], sem_in).wait()

    def issue_out(p, k):
        pltpu.async_copy(o1_v.at[k], o1_h.at[pl.ds(base(p), SEG)], sem_out)
        pltpu.async_copy(o2_v.at[k], o2_h.at[pl.ds(base(p), SEG)], sem_out)
        pltpu.async_copy(o3_v.at[k], o3_h.at[pl.ds(base(p), SEG)], sem_out)

    def wait_out(p, k):
        pltpu.make_async_copy(o1_v.at[k], o1_h.at[pl.ds(base(p), SEG)], sem_out).wait()
        pltpu.make_async_copy(o2_v.at[k], o2_h.at[pl.ds(base(p), SEG)], sem_out).wait()
        pltpu.make_async_copy(o3_v.at[k], o3_h.at[pl.ds(base(p), SEG)], sem_out).wait()

    def compute(p, k):
        magic = (p * NLUT).astype(jnp.float32) + jnp.float32(2.0 ** 23)
        magic_v = jnp.zeros((L,), jnp.float32) + magic

        @plsc.parallel_loop(0, VITER, 1, unroll=8)
        def _(i):
            o = i * L
            x = in_v[k, pl.ds(o, L)]
            f = x * jnp.float32(255.0) + magic_v
            idx = lax.bitcast_convert_type(f, jnp.int32) & jnp.int32(0x3FFF)
            o1_v[k, pl.ds(o, L)] = plsc.load_gather(t1_v, [idx])
            o2_v[k, pl.ds(o, L)] = plsc.load_gather(t2_v, [idx])
            o3_v[k, pl.ds(o, L)] = plsc.load_gather(t3_v, [idx])

    issue_in(0, 0)

    def plane_pair(pp, _):
        for par in (0, 1):
            p = 2 * pp + par
            wait_in(p, par)
            if par == 0:
                issue_in(p + 1, 1)
            else:
                @pl.when(pp < P // 2 - 1)
                def _():
                    issue_in(p + 1, 0)

            @pl.when(pp > 0)
            def _():
                wait_out(p - 2, par)

            compute(p, par)
            issue_out(p, par)
        return 0

    lax.fori_loop(0, P // 2, plane_pair, 0)
    wait_out(P - 2, 0)
    wait_out(P - 1, 1)


@functools.partial(jax.jit)
def _run(imgf, t1, t2, t3):
    out_t = tuple(jax.ShapeDtypeStruct((P * PLANE,), jnp.float32) for _ in range(3))
    scratch = [
        pltpu.VMEM((P * NLUT,), jnp.float32),
        pltpu.VMEM((P * NLUT,), jnp.float32),
        pltpu.VMEM((P * NLUT,), jnp.float32),
        pltpu.VMEM((2, SEG), jnp.float32),
        pltpu.VMEM((2, SEG), jnp.float32),
        pltpu.VMEM((2, SEG), jnp.float32),
        pltpu.VMEM((2, SEG), jnp.float32),
        pltpu.SemaphoreType.DMA,
        pltpu.SemaphoreType.DMA,
    ]
    f = pl.kernel(
        _body, out_type=out_t, mesh=_mesh(), scratch_types=scratch,
        compiler_params=pltpu.CompilerParams(needs_layout_passes=False),
    )
    return f(imgf, t1, t2, t3)


def kernel(img, tf1, tf2, tf3):
    B, C, H, W = img.shape
    o1, o2, o3 = _run(
        img.reshape(P * PLANE),
        tf1.reshape(P * NLUT),
        tf2.reshape(P * NLUT),
        tf3.reshape(P * NLUT),
    )
    shp = (B, C, H, W)
    return (o1.reshape(shp), o2.reshape(shp), o3.reshape(shp))


# speedup vs baseline: 6772.8247x; 3.3602x over previous
import functools

import jax
import jax.numpy as jnp
from jax import lax
from jax.experimental import pallas as pl
from jax.experimental.pallas import tpu as pltpu
from jax.experimental.pallas import tpu_sc as plsc

NC = 2
NS = 16
L = 16
NW = NC * NS

P = 48
H = 512
W = 512
RB = H // NW
SEG = RB * W
NLUT = 256
VITER = SEG // L
CPR = W // L


def _mesh():
    return plsc.VectorSubcoreMesh(
        core_axis_name="c", subcore_axis_name="s", num_cores=NC, num_subcores=NS
    )


def _body(img_h, t1_h, t2_h, t3_h, o1_h, o2_h, o3_h,
          t1_v, t2_v, t3_v, in_v, o1_v, o2_v, o3_v, sem_in, sem_out):
    wid = lax.axis_index("s") * NC + lax.axis_index("c")
    r0 = pl.multiple_of(wid * RB, RB)
    pltpu.async_copy(t1_h, t1_v, sem_in).wait()
    pltpu.async_copy(t2_h, t2_v, sem_in).wait()
    pltpu.async_copy(t3_h, t3_v, sem_in).wait()

    def issue_in(p, k):
        pltpu.async_copy(img_h.at[p, pl.ds(r0, RB)], in_v.at[k], sem_in)

    def wait_in(p, k):
        pltpu.make_async_copy(img_h.at[p, pl.ds(r0, RB)], in_v.at[k], sem_in).wait()

    def issue_out(p, k):
        pltpu.async_copy(o1_v.at[k], o1_h.at[p, pl.ds(r0, RB)], sem_out)
        pltpu.async_copy(o2_v.at[k], o2_h.at[p, pl.ds(r0, RB)], sem_out)
        pltpu.async_copy(o3_v.at[k], o3_h.at[p, pl.ds(r0, RB)], sem_out)

    def wait_out(p, k):
        pltpu.make_async_copy(o1_v.at[k], o1_h.at[p, pl.ds(r0, RB)], sem_out).wait()
        pltpu.make_async_copy(o2_v.at[k], o2_h.at[p, pl.ds(r0, RB)], sem_out).wait()
        pltpu.make_async_copy(o3_v.at[k], o3_h.at[p, pl.ds(r0, RB)], sem_out).wait()

    def compute(p, k):
        magic = (p * NLUT).astype(jnp.float32) + jnp.float32(2.0 ** 23)
        magic_v = jnp.zeros((L,), jnp.float32) + magic

        @plsc.parallel_loop(0, VITER, 1, unroll=8)
        def _(i):
            r = i // CPR
            c = (i % CPR) * L
            x = in_v[k, r, pl.ds(c, L)]
            f = x * jnp.float32(255.0) + magic_v
            idx = lax.bitcast_convert_type(f, jnp.int32) & jnp.int32(0x3FFF)
            o1_v[k, r, pl.ds(c, L)] = plsc.load_gather(t1_v, [idx])
            o2_v[k, r, pl.ds(c, L)] = plsc.load_gather(t2_v, [idx])
            o3_v[k, r, pl.ds(c, L)] = plsc.load_gather(t3_v, [idx])

    issue_in(0, 0)

    def plane_pair(pp, _):
        for par in (0, 1):
            p = 2 * pp + par
            wait_in(p, par)
            if par == 0:
                issue_in(p + 1, 1)
            else:
                @pl.when(pp < P // 2 - 1)
                def _():
                    issue_in(p + 1, 0)

            @pl.when(pp > 0)
            def _():
                wait_out(p - 2, par)

            compute(p, par)
            issue_out(p, par)
        return 0

    lax.fori_loop(0, P // 2, plane_pair, 0)
    wait_out(P - 2, 0)
    wait_out(P - 1, 1)


@functools.partial(jax.jit)
def _run(img3, t1, t2, t3):
    out_t = tuple(jax.ShapeDtypeStruct((P, H, W), jnp.float32) for _ in range(3))
    scratch = [
        pltpu.VMEM((P * NLUT,), jnp.float32),
        pltpu.VMEM((P * NLUT,), jnp.float32),
        pltpu.VMEM((P * NLUT,), jnp.float32),
        pltpu.VMEM((2, RB, W), jnp.float32),
        pltpu.VMEM((2, RB, W), jnp.float32),
        pltpu.VMEM((2, RB, W), jnp.float32),
        pltpu.VMEM((2, RB, W), jnp.float32),
        pltpu.SemaphoreType.DMA,
        pltpu.SemaphoreType.DMA,
    ]
    f = pl.kernel(
        _body, out_type=out_t, mesh=_mesh(), scratch_types=scratch,
        compiler_params=pltpu.CompilerParams(
            needs_layout_passes=False, use_tc_tiling_on_sc=True,
        ),
    )
    return f(img3, t1, t2, t3)


def kernel(img, tf1, tf2, tf3):
    B, C, _, _ = img.shape
    o1, o2, o3 = _run(
        img.reshape(P, H, W),
        tf1.reshape(P * NLUT),
        tf2.reshape(P * NLUT),
        tf3.reshape(P * NLUT),
    )
    shp = (B, C, H, W)
    return (o1.reshape(shp), o2.reshape(shp), o3.reshape(shp))


# speedup vs baseline: 6823.9264x; 1.0075x over previous
import functools

import jax
import jax.numpy as jnp
from jax import lax
from jax.experimental import pallas as pl
from jax.experimental.pallas import tpu as pltpu
from jax.experimental.pallas import tpu_sc as plsc

NC = 2
NS = 16
L = 16
NW = NC * NS

P = 48
H = 512
W = 512
NBUF = 4
RB = 8
BPP = H // (NW * RB)
STEPS = P * BPP
SEG = RB * W
NLUT = 256
VITER = SEG // L
CPR = W // L


def _mesh():
    return plsc.VectorSubcoreMesh(
        core_axis_name="c", subcore_axis_name="s", num_cores=NC, num_subcores=NS
    )


def _body(img_h, t1_h, t2_h, t3_h, o1_h, o2_h, o3_h,
          t1_v, t2_v, t3_v, in_v, o1_v, o2_v, o3_v, *sems):
    sem_in = sems[:NBUF]
    sem_out = sems[NBUF:]
    wid = lax.axis_index("s") * NC + lax.axis_index("c")
    pltpu.async_copy(t1_h, t1_v, sem_in[0]).wait()
    pltpu.async_copy(t2_h, t2_v, sem_in[0]).wait()
    pltpu.async_copy(t3_h, t3_v, sem_in[0]).wait()

    def rows(step):
        p = step // BPP
        r0 = pl.multiple_of((wid * BPP + step % BPP) * RB, RB)
        return p, r0

    def issue_in(step, k):
        p, r0 = rows(step)
        pltpu.async_copy(img_h.at[p, pl.ds(r0, RB)], in_v.at[k], sem_in[k])

    def wait_in(step, k):
        p, r0 = rows(step)
        pltpu.make_async_copy(img_h.at[p, pl.ds(r0, RB)], in_v.at[k],
                              sem_in[k]).wait()

    def issue_out(step, k):
        p, r0 = rows(step)
        pltpu.async_copy(o1_v.at[k], o1_h.at[p, pl.ds(r0, RB)], sem_out[k])
        pltpu.async_copy(o2_v.at[k], o2_h.at[p, pl.ds(r0, RB)], sem_out[k])
        pltpu.async_copy(o3_v.at[k], o3_h.at[p, pl.ds(r0, RB)], sem_out[k])

    def wait_out(step, k):
        p, r0 = rows(step)
        pltpu.make_async_copy(o1_v.at[k], o1_h.at[p, pl.ds(r0, RB)],
                              sem_out[k]).wait()
        pltpu.make_async_copy(o2_v.at[k], o2_h.at[p, pl.ds(r0, RB)],
                              sem_out[k]).wait()
        pltpu.make_async_copy(o3_v.at[k], o3_h.at[p, pl.ds(r0, RB)],
                              sem_out[k]).wait()

    def compute(step, k):
        p = step // BPP
        magic = (p * NLUT).astype(jnp.float32) + jnp.float32(2.0 ** 23)
        magic_v = jnp.zeros((L,), jnp.float32) + magic

        @plsc.parallel_loop(0, VITER, 1, unroll=8)
        def _(i):
            r = i // CPR
            c = (i % CPR) * L
            x = in_v[k, r, pl.ds(c, L)]
            f = x * jnp.float32(255.0) + magic_v
            idx = lax.bitcast_convert_type(f, jnp.int32) & jnp.int32(0x3FFF)
            o1_v[k, r, pl.ds(c, L)] = plsc.load_gather(t1_v, [idx])
            o2_v[k, r, pl.ds(c, L)] = plsc.load_gather(t2_v, [idx])
            o3_v[k, r, pl.ds(c, L)] = plsc.load_gather(t3_v, [idx])

    for s in range(NBUF - 1):
        issue_in(s, s)

    def ring(g, _):
        for par in range(NBUF):
            step = NBUF * g + par
            wait_in(step, par)
            nstep = step + NBUF - 1
            kpre = (par + NBUF - 1) % NBUF

            @pl.when(nstep < STEPS)
            def _():
                issue_in(nstep, kpre)

            @pl.when(g > 0)
            def _():
                wait_out(step - NBUF, par)

            compute(step, par)
            issue_out(step, par)
        return 0

    lax.fori_loop(0, STEPS // NBUF, ring, 0)
    for s in range(NBUF):
        wait_out(STEPS - NBUF + s, s)


@functools.partial(jax.jit)
def _run(img3, t1, t2, t3):
    out_t = tuple(jax.ShapeDtypeStruct((P, H, W), jnp.float32) for _ in range(3))
    scratch = [
        pltpu.VMEM((P * NLUT,), jnp.float32),
        pltpu.VMEM((P * NLUT,), jnp.float32),
        pltpu.VMEM((P * NLUT,), jnp.float32),
        pltpu.VMEM((NBUF, RB, W), jnp.float32),
        pltpu.VMEM((NBUF, RB, W), jnp.float32),
        pltpu.VMEM((NBUF, RB, W), jnp.float32),
        pltpu.VMEM((NBUF, RB, W), jnp.float32),
    ] + [pltpu.SemaphoreType.DMA] * (2 * NBUF)
    f = pl.kernel(
        _body, out_type=out_t, mesh=_mesh(), scratch_types=scratch,
        compiler_params=pltpu.CompilerParams(
            needs_layout_passes=False, use_tc_tiling_on_sc=True,
        ),
    )
    return f(img3, t1, t2, t3)


def kernel(img, tf1, tf2, tf3):
    B, C, _, _ = img.shape
    o1, o2, o3 = _run(
        img.reshape(P, H, W),
        tf1.reshape(P * NLUT),
        tf2.reshape(P * NLUT),
        tf3.reshape(P * NLUT),
    )
    shp = (B, C, H, W)
    return (o1.reshape(shp), o2.reshape(shp), o3.reshape(shp))
